# baseline (speedup 1.0000x reference)
import functools

import jax
import jax.numpy as jnp
from jax import lax
from jax.experimental import pallas as pl
from jax.experimental.pallas import tpu as pltpu
from jax.experimental.pallas import tpu_sc as plsc

_B = 16384
_DIM = 32
_NC = 2
_NS = 16
_NW = _NC * _NS
_BPW = _B // _NW


def _body(user_idx, item_idx, time_idx, dst_u, dst_i, dst_t,
          user_tab, item_tab, time_tab, out,
          idx_u, idx_i, idx_t, odx_u, odx_i, odx_t,
          rows_u, rows_i, rows_t, sem):
    wid = lax.axis_index("s") * _NC + lax.axis_index("c")
    base = wid * _BPW

    pltpu.sync_copy(user_idx.at[pl.ds(base, _BPW)], idx_u)
    pltpu.sync_copy(item_idx.at[pl.ds(base, _BPW)], idx_i)
    pltpu.sync_copy(time_idx.at[pl.ds(base, _BPW)], idx_t)
    pltpu.sync_copy(dst_u.at[pl.ds(base, _BPW)], odx_u)
    pltpu.sync_copy(dst_i.at[pl.ds(base, _BPW)], odx_i)
    pltpu.sync_copy(dst_t.at[pl.ds(base, _BPW)], odx_t)

    cu = pltpu.async_copy(user_tab.at[idx_u], rows_u, sem)
    ci = pltpu.async_copy(item_tab.at[idx_i], rows_i, sem)
    ct = pltpu.async_copy(time_tab.at[idx_t], rows_t, sem)
    cu.wait()
    ci.wait()
    ct.wait()

    su = pltpu.async_copy(rows_u, out.at[odx_u], sem)
    si = pltpu.async_copy(rows_i, out.at[odx_i], sem)
    st = pltpu.async_copy(rows_t, out.at[odx_t], sem)
    su.wait()
    si.wait()
    st.wait()


_emb_call = functools.partial(
    pl.kernel,
    out_type=jax.ShapeDtypeStruct((3 * _B, _DIM), jnp.float32),
    mesh=plsc.VectorSubcoreMesh(core_axis_name="c", subcore_axis_name="s"),
    compiler_params=pltpu.CompilerParams(use_tc_tiling_on_sc=False),
    scratch_types=[
        pltpu.VMEM((_BPW,), jnp.int32),
        pltpu.VMEM((_BPW,), jnp.int32),
        pltpu.VMEM((_BPW,), jnp.int32),
        pltpu.VMEM((_BPW,), jnp.int32),
        pltpu.VMEM((_BPW,), jnp.int32),
        pltpu.VMEM((_BPW,), jnp.int32),
        pltpu.VMEM((_BPW, _DIM), jnp.float32),
        pltpu.VMEM((_BPW, _DIM), jnp.float32),
        pltpu.VMEM((_BPW, _DIM), jnp.float32),
        pltpu.SemaphoreType.DMA,
    ],
)(_body)


@jax.jit
def kernel(user_idx, item_idx, time_idx, user_table, item_table, time_table):
    row3 = jnp.arange(_B, dtype=jnp.int32) * 3
    out3 = _emb_call(user_idx, item_idx, time_idx,
                     row3, row3 + 1, row3 + 2,
                     user_table, item_table, time_table)
    return out3.reshape(_B, 3 * _DIM)
